# baseline (speedup 1.0000x reference)
import functools

import jax
import jax.numpy as jnp
from jax import lax
from jax.experimental import pallas as pl
from jax.experimental.pallas import tpu as pltpu
from jax.experimental.pallas import tpu_sc as plsc

_ROWS = 32
_N = 32768
_K = 16384
_L = 16
_NV = _N // _L
_U = 8
_SCALE = 10000.0
_TEMP = 0.3333333333333333
_C2 = -(_SCALE / _TEMP)

_NB = 1024
_HSTRIDE = _NB + 1
_HIST = 16512


def _build():
  mesh = plsc.VectorSubcoreMesh(core_axis_name="c", subcore_axis_name="s")

  @functools.partial(
      pl.kernel,
      mesh=mesh,
      compiler_params=pltpu.CompilerParams(needs_layout_passes=False),
      out_type=jax.ShapeDtypeStruct((_ROWS, _N), jnp.float32),
      scratch_types=[
          pltpu.VMEM((_N,), jnp.float32),
          pltpu.VMEM((_N,), jnp.float32),
          pltpu.VMEM((_HIST,), jnp.int32),
          pltpu.SemaphoreType.DMA,
      ],
  )
  def k(z_hbm, out_hbm, buf, sbuf, hist, sem):
    wid = lax.axis_index("s") * 2 + lax.axis_index("c")
    lane_base = lax.iota(jnp.int32, _L) * _HSTRIDE
    zeros_i = jnp.zeros((_L,), jnp.int32)
    ones_i = jnp.ones((_L,), jnp.int32)

    copy_in = pltpu.async_copy(z_hbm.at[wid], buf, sem)

    def zinit(i, c):
      for j in range(_U):
        hist[pl.ds(i * (_L * _U) + j * _L, _L)] = zeros_i
      return c

    lax.fori_loop(0, _HIST // (_L * _U), zinit, 0)
    copy_in.wait()

    _U1 = 16

    def p1(i, c):
      zs = [buf[pl.ds(i * (_L * _U1) + j * _L, _L)] for j in range(_U1)]
      ss = [1.0 / (1.0 + jnp.exp(z * _C2)) for z in zs]
      for j in range(_U1):
        sbuf[pl.ds(i * (_L * _U1) + j * _L, _L)] = ss[j]
      keys = [lax.bitcast_convert_type(s, jnp.int32) for s in ss]
      addrs = [lane_base + lax.shift_right_logical(kk, 20) for kk in keys]
      for j in range(_U1):
        plsc.addupdate_scatter(hist, [addrs[j]], ones_i)
      return c

    lax.fori_loop(0, _NV // _U1, p1, 0)

    def fold(target):
      def mbody(j, carry):
        run, bvec, belowvec = carry
        tot = zeros_i
        for l in range(_L):
          sl = pl.ds(l * _HSTRIDE + j * _L, _L)
          tot = tot + hist[sl]
          hist[sl] = zeros_i
        cum = jnp.cumsum(tot) + run
        ind = (cum < target).astype(jnp.int32)
        bvec = bvec + ind
        belowvec = belowvec + tot * ind
        run = run + jnp.sum(tot)
        return run, bvec, belowvec

      init = (jnp.int32(0), zeros_i, zeros_i)
      _, bvec, belowvec = lax.fori_loop(0, _NB // _L, mbody, init)
      return jnp.sum(bvec), jnp.sum(belowvec)

    b1, below1 = fold(jnp.int32(_K))
    t2 = _K - below1

    _U2 = 8

    def p2(i, c):
      ks = [
          lax.bitcast_convert_type(
              sbuf[pl.ds(i * (_L * _U2) + j * _L, _L)], jnp.int32
          )
          for j in range(_U2)
      ]
      b1s = b1 * (1 << 20)
      rr = [lax.shift_right_logical(kk - b1s, 10) for kk in ks]
      addrs = [lane_base + jnp.minimum(r, _NB) for r in rr]
      for j in range(_U2):
        plsc.addupdate_scatter(hist, [addrs[j]], ones_i)
      return c

    lax.fori_loop(0, _NV // _U2, p2, 0)
    b2, below2 = fold(t2)
    t3 = t2 - below2
    pref = b1 * _NB + b2

    def p3(i, c):
      ks = [
          lax.bitcast_convert_type(
              sbuf[pl.ds(i * (_L * _U2) + j * _L, _L)], jnp.int32
          )
          for j in range(_U2)
      ]
      prefs = pref * (1 << 10)
      r2u = [
          lax.bitcast_convert_type(kk - prefs, jnp.uint32) for kk in ks
      ]
      addrs = [
          lane_base
          + lax.bitcast_convert_type(
              jnp.minimum(r2, jnp.uint32(_NB)), jnp.int32
          )
          for r2 in r2u
      ]
      for j in range(_U2):
        plsc.addupdate_scatter(hist, [addrs[j]], ones_i)
      return c

    lax.fori_loop(0, _NV // _U2, p3, 0)
    b3, below3 = fold(t3)
    thr = pref * _NB + b3
    r = t3 - below3

    _NCH = 1
    _CVR = _NV // _NCH
    _CEL = _CVR * _L

    _U4 = 8

    def p4(i, rc):
      ss = [sbuf[pl.ds(i * (_L * _U4) + j * _L, _L)] for j in range(_U4)]
      ks = [lax.bitcast_convert_type(s, jnp.int32) for s in ss]
      eqm = [kk == thr for kk in ks]
      csum = [jnp.cumsum(m.astype(jnp.int32)) for m in eqm]
      cnts = [plsc.all_reduce_population_count(m) for m in eqm]
      outs = []
      for j in range(_U4):
        pos = csum[j] + rc
        zero = (ks[j] < thr) | (eqm[j] & (pos <= r))
        outs.append(jnp.where(zero, 0.0, ss[j]))
        rc = rc + cnts[j]
      for j in range(_U4):
        buf[pl.ds(i * (_L * _U4) + j * _L, _L)] = outs[j]
      return rc

    rc = zeros_i
    copies = []
    for ch in range(_NCH):
      lo, hi = ch * (_CVR // _U4), (ch + 1) * (_CVR // _U4)
      rc = lax.fori_loop(lo, hi, p4, rc)
      copies.append(
          pltpu.async_copy(
              buf.at[pl.ds(ch * _CEL, _CEL)],
              out_hbm.at[wid, pl.ds(ch * _CEL, _CEL)],
              sem,
          )
      )
    for cp in copies:
      cp.wait()

  return k


_mask_sc = _build()


def kernel(z_loga):
  return _mask_sc(z_loga)


# speedup vs baseline: 1.0524x; 1.0524x over previous
import functools

import jax
import jax.numpy as jnp
from jax import lax
from jax.experimental import pallas as pl
from jax.experimental.pallas import tpu as pltpu
from jax.experimental.pallas import tpu_sc as plsc

_ROWS = 32
_N = 32768
_K = 16384
_L = 16
_NV = _N // _L
_U = 8
_SCALE = 10000.0
_TEMP = 0.3333333333333333
_C2 = -(_SCALE / _TEMP)

_NB = 1024
_HSTRIDE = _NB + 1
_HIST = 16512


def _build():
  mesh = plsc.VectorSubcoreMesh(core_axis_name="c", subcore_axis_name="s")

  @functools.partial(
      pl.kernel,
      mesh=mesh,
      compiler_params=pltpu.CompilerParams(needs_layout_passes=False),
      out_type=jax.ShapeDtypeStruct((_ROWS, _N), jnp.float32),
      scratch_types=[
          pltpu.VMEM((_N,), jnp.float32),
          pltpu.VMEM((_N,), jnp.float32),
          pltpu.VMEM((_HIST,), jnp.int32),
          pltpu.SemaphoreType.DMA,
      ],
  )
  def k(z_hbm, out_hbm, buf, sbuf, hist, sem):
    wid = lax.axis_index("s") * 2 + lax.axis_index("c")
    lane_base = lax.iota(jnp.int32, _L) * _HSTRIDE
    zeros_i = jnp.zeros((_L,), jnp.int32)
    ones_i = jnp.ones((_L,), jnp.int32)

    copy_in = pltpu.async_copy(z_hbm.at[wid], buf, sem)

    def zinit(i, c):
      for j in range(_U):
        hist[pl.ds(i * (_L * _U) + j * _L, _L)] = zeros_i
      return c

    lax.fori_loop(0, _HIST // (_L * _U), zinit, 0)
    copy_in.wait()

    _U1 = 16

    def p1(i, c):
      zs = [buf[pl.ds(i * (_L * _U1) + j * _L, _L)] for j in range(_U1)]
      ss = [1.0 / (1.0 + jnp.exp(z * _C2)) for z in zs]
      for j in range(_U1):
        sbuf[pl.ds(i * (_L * _U1) + j * _L, _L)] = ss[j]
      keys = [lax.bitcast_convert_type(s, jnp.int32) for s in ss]
      addrs = [lane_base + lax.shift_right_logical(kk, 20) for kk in keys]
      for j in range(_U1):
        plsc.addupdate_scatter(hist, [addrs[j]], ones_i)
      return c

    lax.fori_loop(0, _NV // _U1, p1, 0)

    def fold(target):
      def mbody(j, carry):
        run, bvec, belowvec = carry
        tot = zeros_i
        for l in range(_L):
          sl = pl.ds(l * _HSTRIDE + j * _L, _L)
          tot = tot + hist[sl]
          hist[sl] = zeros_i
        cum = jnp.cumsum(tot) + run
        ind = (cum < target).astype(jnp.int32)
        bvec = bvec + ind
        belowvec = belowvec + tot * ind
        run = run + jnp.sum(tot)
        return run, bvec, belowvec

      init = (jnp.int32(0), zeros_i, zeros_i)
      _, bvec, belowvec = lax.fori_loop(0, _NB // _L, mbody, init)
      return jnp.sum(bvec), jnp.sum(belowvec)

    b1, below1 = fold(jnp.int32(_K))
    t2 = _K - below1

    _U2 = 16

    def p2(i, c):
      ks = [
          lax.bitcast_convert_type(
              sbuf[pl.ds(i * (_L * _U2) + j * _L, _L)], jnp.int32
          )
          for j in range(_U2)
      ]
      b1s = b1 * (1 << 20)
      rr = [lax.shift_right_logical(kk - b1s, 10) for kk in ks]
      addrs = [lane_base + jnp.minimum(r, _NB) for r in rr]
      for j in range(_U2):
        plsc.addupdate_scatter(hist, [addrs[j]], ones_i)
      return c

    lax.fori_loop(0, _NV // _U2, p2, 0)
    b2, below2 = fold(t2)
    t3 = t2 - below2
    pref = b1 * _NB + b2

    def p3(i, c):
      ks = [
          lax.bitcast_convert_type(
              sbuf[pl.ds(i * (_L * _U2) + j * _L, _L)], jnp.int32
          )
          for j in range(_U2)
      ]
      prefs = pref * (1 << 10)
      r2u = [
          lax.bitcast_convert_type(kk - prefs, jnp.uint32) for kk in ks
      ]
      addrs = [
          lane_base
          + lax.bitcast_convert_type(
              jnp.minimum(r2, jnp.uint32(_NB)), jnp.int32
          )
          for r2 in r2u
      ]
      for j in range(_U2):
        plsc.addupdate_scatter(hist, [addrs[j]], ones_i)
      return c

    lax.fori_loop(0, _NV // _U2, p3, 0)
    b3, below3 = fold(t3)
    thr = pref * _NB + b3
    r = t3 - below3

    _NCH = 2
    _CVR = _NV // _NCH
    _CEL = _CVR * _L

    _U4 = 8

    def p4(i, rc):
      ss = [sbuf[pl.ds(i * (_L * _U4) + j * _L, _L)] for j in range(_U4)]
      ks = [lax.bitcast_convert_type(s, jnp.int32) for s in ss]
      eqm = [kk == thr for kk in ks]
      csum = [jnp.cumsum(m.astype(jnp.int32)) for m in eqm]
      cnts = [plsc.all_reduce_population_count(m) for m in eqm]
      outs = []
      for j in range(_U4):
        pos = csum[j] + rc
        zero = (ks[j] < thr) | (eqm[j] & (pos <= r))
        outs.append(jnp.where(zero, 0.0, ss[j]))
        rc = rc + cnts[j]
      for j in range(_U4):
        buf[pl.ds(i * (_L * _U4) + j * _L, _L)] = outs[j]
      return rc

    rc = zeros_i
    copies = []
    for ch in range(_NCH):
      lo, hi = ch * (_CVR // _U4), (ch + 1) * (_CVR // _U4)
      rc = lax.fori_loop(lo, hi, p4, rc)
      copies.append(
          pltpu.async_copy(
              buf.at[pl.ds(ch * _CEL, _CEL)],
              out_hbm.at[wid, pl.ds(ch * _CEL, _CEL)],
              sem,
          )
      )
    for cp in copies:
      cp.wait()

  return k


_mask_sc = _build()


def kernel(z_loga):
  return _mask_sc(z_loga)
